# baseline (speedup 1.0000x reference)
import functools
import math

import jax
import jax.numpy as jnp
from jax import lax
from jax.experimental import pallas as pl
from jax.experimental.pallas import tpu as pltpu
from jax.experimental.pallas import tpu_sc as plsc

N = 10000
E = 320000
B = 64
D = 128
NF = 64
H = 4
MAXEL = 100

NP = 10240
NW = 32
CH = 512
NCHUNK = E // CH
NWN = NP // NW
TE = 512
TN = 2048
INV_SQRT_HD = 1.0 / math.sqrt(D // H)
TWO_PI = 2.0 * math.pi
F32 = jnp.float32

_SC_MESH = plsc.VectorSubcoreMesh(core_axis_name="c", subcore_axis_name="s")


def _nchunks(wid, nchunk):
    rem = nchunk - NW * (nchunk // NW)
    return jnp.where(wid < rem, nchunk // NW + 1, nchunk // NW)


def _dotT(a, b):
    return lax.dot_general(a, b, (((0,), (0,)), ((), ())),
                           preferred_element_type=F32)


def _dot(a, b):
    return jnp.dot(a, b, preferred_element_type=F32)


def _dotp(a, b):
    return jnp.dot(a, b, preferred_element_type=F32,
                   precision=lax.Precision.HIGHEST)


def _ln_tc(x, g, b):
    m = jnp.mean(x, axis=-1, keepdims=True)
    v = jnp.mean((x - m) ** 2, axis=-1, keepdims=True)
    return (x - m) / jnp.sqrt(v + 1e-5) * g + b


def _prep_body(emb_ref, nw1_ref, lat_ref, nw2_ref, nb_ref, a_ref, bl_ref):
    a_ref[...] = _dot(emb_ref[...], nw1_ref[...])
    lat = lat_ref[...]
    cols = []
    for (i, k) in ((0, 0), (0, 1), (0, 2), (1, 1), (1, 2), (2, 2)):
        c = (lat[:, 0 + i:1 + i] * lat[:, 0 + k:1 + k]
             + lat[:, 3 + i:4 + i] * lat[:, 3 + k:4 + k]
             + lat[:, 6 + i:7 + i] * lat[:, 6 + k:7 + k])
        cols.append(c)
    lf = jnp.concatenate(cols, axis=1)
    bl_ref[...] = _dot(lf, nw2_ref[...]) + nb_ref[...]


def _prep_call(embP, nw1, lat9, nw2, nb):
    return pl.pallas_call(
        _prep_body,
        out_shape=(jax.ShapeDtypeStruct((D, D), F32),
                   jax.ShapeDtypeStruct((B, D), F32)),
    )(embP, nw1, lat9, nw2, nb)


def _k1_body(typesP, batchP, a_t, bl_t, fc_flat, src, dst, shifts, hn0, disp_flat,
             idxn_v, rows_a, rows_b, fc_v, idxs_v, idxd_v, sh_v, out_v, sem):
    cc = lax.axis_index("c")
    ss = lax.axis_index("s")
    wid = cc * 16 + ss

    nbase = wid * NWN
    pltpu.sync_copy(typesP.at[pl.ds(nbase, NWN)], idxn_v)
    pltpu.async_copy(a_t.at[idxn_v], rows_a, sem).wait()
    pltpu.sync_copy(batchP.at[pl.ds(nbase, NWN)], idxn_v)
    pltpu.async_copy(bl_t.at[idxn_v], rows_b, sem).wait()

    def nbody(r, carry):
        for kk in range(D // 16):
            sl = pl.ds(kk * 16, 16)
            rows_a[r, sl] = rows_a[r, sl] + rows_b[r, sl]
        return carry

    lax.fori_loop(0, NWN, nbody, 0)
    pltpu.sync_copy(rows_a, hn0.at[pl.ds(nbase, NWN), :])

    pltpu.sync_copy(fc_flat, fc_v)
    z16 = jnp.zeros((16,), F32)

    def zout(kk, carry):
        out_v[pl.ds(kk * 16, 16)] = z16
        return carry

    lax.fori_loop(0, (CH * 16) // 16, zout, 0)
    iota = lax.iota(jnp.int32, 16)

    def ebody(i, carry):
        base = (wid + i * NW) * CH
        pltpu.sync_copy(src.at[pl.ds(base, CH)], idxs_v)
        pltpu.sync_copy(dst.at[pl.ds(base, CH)], idxd_v)
        pltpu.sync_copy(shifts.at[pl.ds(base * 3, CH * 3)], sh_v)

        def jbody(j, jcarry):
            sl = pl.ds(j * 16, 16)
            s16 = idxs_v[sl]
            d16 = idxd_v[sl]
            e16 = j * 16 + iota
            for comp in range(3):
                fs = plsc.load_gather(fc_v, [s16 * 3 + comp])
                fd = plsc.load_gather(fc_v, [d16 * 3 + comp])
                sh = plsc.load_gather(sh_v, [e16 * 3 + comp])
                dv = fd - fs + sh
                ti = dv.astype(jnp.int32).astype(F32)
                fl = jnp.where(dv < ti, ti - 1.0, ti)
                plsc.store_scatter(out_v, [e16 * 16 + comp], dv - fl)
            return jcarry

        lax.fori_loop(0, CH // 16, jbody, 0)
        pltpu.sync_copy(out_v, disp_flat.at[pl.ds(base * 16, CH * 16)])
        return carry

    lax.fori_loop(0, _nchunks(wid, NCHUNK), ebody, 0)


def _k1_call(typesP, batchP, a_t, bl_t, fcp, src, dst, shifts):
    return pl.kernel(
        _k1_body,
        out_type=(jax.ShapeDtypeStruct((NP, D), F32),
                  jax.ShapeDtypeStruct((E * 16,), F32)),
        mesh=_SC_MESH,
        compiler_params=pltpu.CompilerParams(needs_layout_passes=False),
        scratch_types=[
            pltpu.VMEM((NWN,), jnp.int32),
            pltpu.VMEM((NWN, D), F32),
            pltpu.VMEM((NWN, D), F32),
            pltpu.VMEM((N * 3,), F32),
            pltpu.VMEM((CH,), jnp.int32),
            pltpu.VMEM((CH,), jnp.int32),
            pltpu.VMEM((CH * 3,), F32),
            pltpu.VMEM((CH * 16,), F32),
            pltpu.SemaphoreType.DMA,
        ],
    )(typesP, batchP, a_t, bl_t, fcp, src, dst, shifts)


_SIN_C = (6.2831855, -41.3417, 81.60525, -76.70578, 42.057533, -15.085474,
          3.7785523, -0.6179781)
_COS_C = (1.0, -19.739208, 64.93939, -85.45682, 60.244595, -26.425692,
          7.8995357, -1.6978502, 0.24478738)


def _k2_body(disp_ref, e16_ref, fint_ref, w1_ref, w2_ref, eb_ref, he_ref):
    d16 = disp_ref[...]
    f0 = _dot(d16, e16_ref[...])
    x = f0 * fint_ref[...]
    t = (x + 0.5).astype(jnp.int32).astype(F32)
    u = x - t
    v = u * u
    sp = jnp.full_like(v, _SIN_C[-1])
    for c in _SIN_C[-2::-1]:
        sp = sp * v + c
    sp = sp * u
    cp = jnp.full_like(v, _COS_C[-1])
    for c in _COS_C[-2::-1]:
        cp = cp * v + c
    he_ref[...] = (_dot(sp, w1_ref[...]) + _dot(cp, w2_ref[...]) + eb_ref[...])


def _k2_call(dispP, e16, fint, w1, w2, eb):
    return pl.pallas_call(
        _k2_body,
        grid=(E // TE,),
        in_specs=[
            pl.BlockSpec((TE, 16), lambda i: (i, 0)),
            pl.BlockSpec((16, 3 * NF), lambda i: (0, 0)),
            pl.BlockSpec((1, 3 * NF), lambda i: (0, 0)),
            pl.BlockSpec((3 * NF, D), lambda i: (0, 0)),
            pl.BlockSpec((3 * NF, D), lambda i: (0, 0)),
            pl.BlockSpec((1, D), lambda i: (0, 0)),
        ],
        out_specs=pl.BlockSpec((TE, D), lambda i: (i, 0)),
        out_shape=jax.ShapeDtypeStruct((E, D), F32),
    )(dispP, e16, fint, w1, w2, eb)


CH3 = 256
NCHUNK3 = E // CH3


def _k3_body(hn, src, dst, hs, hd, idxs_v, idxd_v, rows_s, rows_d,
             sg0, sg1, sw0, sw1):
    wid = lax.axis_index("c") * 16 + lax.axis_index("s")

    def body(i, carry):
        base = (wid + i * NW) * CH3
        pltpu.sync_copy(src.at[pl.ds(base, CH3)], idxs_v)
        pltpu.sync_copy(dst.at[pl.ds(base, CH3)], idxd_v)
        g0 = pltpu.async_copy(hn.at[idxs_v], rows_s, sg0)
        g1 = pltpu.async_copy(hn.at[idxd_v], rows_d, sg1)
        g0.wait()
        w0 = pltpu.async_copy(rows_s, hs.at[pl.ds(base, CH3), :], sw0)
        g1.wait()
        w1 = pltpu.async_copy(rows_d, hd.at[pl.ds(base, CH3), :], sw1)
        w0.wait()
        w1.wait()
        return carry

    lax.fori_loop(0, _nchunks(wid, NCHUNK3), body, 0)


def _k3_call(hn, src, dst):
    return pl.kernel(
        _k3_body,
        out_type=(jax.ShapeDtypeStruct((E, D), F32),
                  jax.ShapeDtypeStruct((E, D), F32)),
        mesh=_SC_MESH,
        scratch_types=[
            pltpu.VMEM((CH3,), jnp.int32),
            pltpu.VMEM((CH3,), jnp.int32),
            pltpu.VMEM((CH3, D), F32),
            pltpu.VMEM((CH3, D), F32),
            pltpu.SemaphoreType.DMA,
            pltpu.SemaphoreType.DMA,
            pltpu.SemaphoreType.DMA,
            pltpu.SemaphoreType.DMA,
        ],
    )(hn, src, dst)


def _k4_body(hs_ref, hd_ref, he_ref, ssel_ref, wq1, wq2, bq, wk1, wk2, bk,
             wv1, wv2, bv, w_ref, ex_ref):
    hs = hs_ref[...]
    hd = hd_ref[...]
    he = he_ref[...]
    q = _dot(hs, wq1[...]) + _dot(he, wq2[...]) + bq[...]
    k = _dot(hd, wk1[...]) + _dot(he, wk2[...]) + bk[...]
    v = _dot(hd, wv1[...]) + _dot(he, wv2[...]) + bv[...]
    qk = q * k
    sb = _dot(qk, ssel_ref[...])
    exb = jnp.exp(jnp.minimum(sb, 60.0))
    w_ref[...] = exb * v
    ex_ref[...] = exb


def _k4_call(hs, hd, he, ssel, wq1, wq2, bq, wk1, wk2, bk, wv1, wv2, bv):
    wspec = pl.BlockSpec((D, D), lambda i: (0, 0))
    bspec = pl.BlockSpec((1, D), lambda i: (0, 0))
    espec = pl.BlockSpec((TE, D), lambda i: (i, 0))
    return pl.pallas_call(
        _k4_body,
        grid=(E // TE,),
        in_specs=[espec, espec, espec, wspec,
                  wspec, wspec, bspec, wspec, wspec, bspec,
                  wspec, wspec, bspec],
        out_specs=(pl.BlockSpec((TE, D), lambda i: (i, 0)),
                   pl.BlockSpec((TE, D), lambda i: (i, 0))),
        out_shape=(jax.ShapeDtypeStruct((E, D), F32),
                   jax.ShapeDtypeStruct((E, D), F32)),
    )(hs, hd, he, ssel, wq1, wq2, bq, wk1, wk2, bk, wv1, wv2, bv)


CH5 = 256
NCHUNK5 = E // CH5


def _k5_body(w, x2, src, u, dens, idx_v, idxr_v, w_v, acc, sem):
    cc = lax.axis_index("c")
    ss = lax.axis_index("s")
    wid = cc * 16 + ss
    z16 = jnp.zeros((16,), F32)
    iota = lax.iota(jnp.int32, 16)

    def _zero_wv():
        def zrow(r, carry):
            for kk in range(D // 16):
                w_v[r, pl.ds(kk * 16, 16)] = z16
            return carry

        lax.fori_loop(0, 128, zrow, 0)

    def _set_own_rows(kk):
        row0 = ss * (NP // 16) + kk * 128

        def seti(j, c2):
            idxr_v[pl.ds(j * 16, 16)] = row0 + j * 16 + iota
            return c2

        lax.fori_loop(0, 128 // 16, seti, 0)
        return row0

    def _zero_acc():
        def initkk(kk, carry):
            _set_own_rows(kk)
            pltpu.sync_copy(w_v.at[pl.ds(0, 128), :], acc.at[idxr_v])
            return carry

        lax.fori_loop(0, NP // 16 // 128, initkk, 0)

    def _accum(ref):
        def body(i, carry):
            base = (wid + i * NW) * CH5
            pltpu.sync_copy(src.at[pl.ds(base, CH5)], idx_v)
            pltpu.sync_copy(ref.at[pl.ds(base, CH5), :], w_v)
            pltpu.sync_copy(w_v, acc.at[idx_v], add=True)
            return carry

        lax.fori_loop(0, _nchunks(wid, NCHUNK5), body, 0)

    def _readback(out3):
        def outkk(kk, carry):
            row0 = _set_own_rows(kk)
            pltpu.async_copy(acc.at[idxr_v], w_v.at[pl.ds(0, 128), :],
                             sem).wait()
            pltpu.sync_copy(w_v.at[pl.ds(0, 128), :],
                            out3.at[cc, pl.ds(row0, 128), :])
            return carry

        lax.fori_loop(0, NP // 16 // 128, outkk, 0)

    _zero_wv()
    _zero_acc()
    plsc.subcore_barrier()
    _accum(w)
    plsc.subcore_barrier()
    _readback(u)
    _zero_wv()
    _zero_acc()
    plsc.subcore_barrier()
    _accum(x2)
    plsc.subcore_barrier()
    _readback(dens)


def _k5_call(w, x2, src):
    return pl.kernel(
        _k5_body,
        out_type=(jax.ShapeDtypeStruct((2, NP, D), F32),
                  jax.ShapeDtypeStruct((2, NP, D), F32)),
        mesh=_SC_MESH,
        compiler_params=pltpu.CompilerParams(needs_layout_passes=False),
        scratch_types=[
            pltpu.VMEM((CH5,), jnp.int32),
            pltpu.VMEM((128,), jnp.int32),
            pltpu.VMEM((CH5, D), F32),
            pltpu.VMEM_SHARED((NP, D), F32),
            pltpu.SemaphoreType.DMA,
        ],
    )(w, x2, src)


def _k6_body(hn_ref, u0_ref, u1_ref, d0_ref, d1_ref, ow, ob, n1g, n1b,
             f1, f1b, f2, f2b, n2g, n2b, out_ref):
    denb = d0_ref[...] + d1_ref[...]
    agg = (u0_ref[...] + u1_ref[...]) / (denb + 1e-16)
    o = _dotp(agg, ow[...]) + ob[...]
    x = hn_ref[...] + o
    h1 = _ln_tc(x, n1g[...], n1b[...])
    ffh = _dotp(h1, f1[...]) + f1b[...]
    gg = 0.5 * ffh * (1.0 + lax.erf(ffh * (1.0 / math.sqrt(2.0))))
    ff = _dotp(gg, f2[...]) + f2b[...]
    out_ref[...] = _ln_tc(h1 + ff, n2g[...], n2b[...])


def _k6_call(hn, u2, dens2, ow, ob, n1g, n1b, f1, f1b, f2, f2b, n2g, n2b):
    nspec = pl.BlockSpec((TN, D), lambda i: (i, 0))
    return pl.pallas_call(
        _k6_body,
        grid=(NP // TN,),
        in_specs=[
            nspec,
            pl.BlockSpec((TN, D), lambda i: (i, 0)),
            pl.BlockSpec((TN, D), lambda i: (i + NP // TN, 0)),
            pl.BlockSpec((TN, D), lambda i: (i, 0)),
            pl.BlockSpec((TN, D), lambda i: (i + NP // TN, 0)),
            pl.BlockSpec((D, D), lambda i: (0, 0)),
            pl.BlockSpec((1, D), lambda i: (0, 0)),
            pl.BlockSpec((1, D), lambda i: (0, 0)),
            pl.BlockSpec((1, D), lambda i: (0, 0)),
            pl.BlockSpec((D, 4 * D), lambda i: (0, 0)),
            pl.BlockSpec((1, 4 * D), lambda i: (0, 0)),
            pl.BlockSpec((4 * D, D), lambda i: (0, 0)),
            pl.BlockSpec((1, D), lambda i: (0, 0)),
            pl.BlockSpec((1, D), lambda i: (0, 0)),
            pl.BlockSpec((1, D), lambda i: (0, 0)),
        ],
        out_specs=nspec,
        out_shape=jax.ShapeDtypeStruct((NP, D), F32),
    )(hn, u2, u2, dens2, dens2, ow, ob, n1g, n1b, f1, f1b, f2, f2b, n2g, n2b)


def kernel(atom_types, lattices, batch_idx, frac_coords, edge_index,
           edge_unit_shifts, num_atoms, token_idx, atom_emb, node_w, node_b,
           edge_w, edge_b, wq_w, wq_b, wkv_w, wkv_b, out_w, out_b, n1_g, n1_b,
           f1_w, f1_b, f2_w, f2_b, n2_g, n2_b):
    src = edge_index[0].astype(jnp.int32)
    dst = edge_index[1].astype(jnp.int32)
    typesP = jnp.pad(atom_types.astype(jnp.int32), (0, NP - N))
    batchP = jnp.pad(batch_idx.astype(jnp.int32), (0, NP - N))
    fcp = frac_coords.astype(F32).reshape(-1)
    shifts = edge_unit_shifts.astype(F32).reshape(-1)
    embP = jnp.pad(atom_emb.astype(F32), ((0, D - MAXEL), (0, 0)))
    lat9 = lattices.astype(F32).reshape(B, 9)

    cols = jnp.arange(3 * NF, dtype=jnp.int32)
    e16 = (cols[None, :] // NF == jnp.arange(16, dtype=jnp.int32)[:, None]
           ).astype(F32)
    fint = (cols % NF).astype(F32)[None, :]
    dcol = jnp.arange(D, dtype=jnp.int32)
    ssel = jnp.where(dcol[:, None] // 32 == dcol[None, :] // 32,
                     INV_SQRT_HD, 0.0).astype(F32)

    a_t, bl_t = _prep_call(embP, node_w[:D], lat9, node_w[D:], node_b[None, :])
    hn0, disp_flat = _k1_call(typesP, batchP, a_t, bl_t, fcp, src, dst, shifts)
    h_e = _k2_call(disp_flat.reshape(E, 16), e16, fint, edge_w[:3 * NF],
                   edge_w[3 * NF:], edge_b[None, :])

    hn = hn0
    for li in range(2):
        hs_, hd_ = _k3_call(hn, src, dst)
        w_, exp_ = _k4_call(
            hs_, hd_, h_e, ssel,
            wq_w[li, :D], wq_w[li, D:], wq_b[li][None, :],
            wkv_w[li, :D, :D], wkv_w[li, D:, :D], wkv_b[li][None, :D],
            wkv_w[li, :D, D:], wkv_w[li, D:, D:], wkv_b[li][None, D:])
        u_, dens_ = _k5_call(w_, exp_, src)
        hn = _k6_call(
            hn, u_.reshape(2 * NP, D), dens_.reshape(2 * NP, D),
            out_w[li], out_b[li][None, :], n1_g[li][None, :], n1_b[li][None, :],
            f1_w[li], f1_b[li][None, :], f2_w[li], f2_b[li][None, :],
            n2_g[li][None, :], n2_b[li][None, :])
    return hn[:N]


# speedup vs baseline: 1.0067x; 1.0067x over previous
import functools
import math

import jax
import jax.numpy as jnp
from jax import lax
from jax.experimental import pallas as pl
from jax.experimental.pallas import tpu as pltpu
from jax.experimental.pallas import tpu_sc as plsc

N = 10000
E = 320000
B = 64
D = 128
NF = 64
H = 4
MAXEL = 100

NP = 10240
NW = 32
CH = 512
NCHUNK = E // CH
NWN = NP // NW
TE = 512
TN = 2048
INV_SQRT_HD = 1.0 / math.sqrt(D // H)
TWO_PI = 2.0 * math.pi
F32 = jnp.float32

_SC_MESH = plsc.VectorSubcoreMesh(core_axis_name="c", subcore_axis_name="s")


def _nchunks(wid, nchunk):
    rem = nchunk - NW * (nchunk // NW)
    return jnp.where(wid < rem, nchunk // NW + 1, nchunk // NW)


def _dotT(a, b):
    return lax.dot_general(a, b, (((0,), (0,)), ((), ())),
                           preferred_element_type=F32)


def _dot(a, b):
    return jnp.dot(a, b, preferred_element_type=F32)


def _dotp(a, b):
    return jnp.dot(a, b, preferred_element_type=F32,
                   precision=lax.Precision.HIGHEST)


def _ln_tc(x, g, b):
    m = jnp.mean(x, axis=-1, keepdims=True)
    v = jnp.mean((x - m) ** 2, axis=-1, keepdims=True)
    return (x - m) / jnp.sqrt(v + 1e-5) * g + b


def _prep_body(emb_ref, nw1_ref, lat_ref, nw2_ref, nb_ref, a_ref, bl_ref):
    a_ref[...] = _dot(emb_ref[...], nw1_ref[...])
    lat = lat_ref[...]
    cols = []
    for (i, k) in ((0, 0), (0, 1), (0, 2), (1, 1), (1, 2), (2, 2)):
        c = (lat[:, 0 + i:1 + i] * lat[:, 0 + k:1 + k]
             + lat[:, 3 + i:4 + i] * lat[:, 3 + k:4 + k]
             + lat[:, 6 + i:7 + i] * lat[:, 6 + k:7 + k])
        cols.append(c)
    lf = jnp.concatenate(cols, axis=1)
    bl_ref[...] = _dot(lf, nw2_ref[...]) + nb_ref[...]


def _prep_call(embP, nw1, lat9, nw2, nb):
    return pl.pallas_call(
        _prep_body,
        out_shape=(jax.ShapeDtypeStruct((D, D), F32),
                   jax.ShapeDtypeStruct((B, D), F32)),
    )(embP, nw1, lat9, nw2, nb)


def _k1_body(typesP, batchP, a_t, bl_t, fc_flat, src, dst, shifts, hn0, disp_flat,
             idxn_v, rows_a, rows_b, fc_v, idxs_v, idxd_v, sh_v, out_v, sem):
    cc = lax.axis_index("c")
    ss = lax.axis_index("s")
    wid = cc * 16 + ss

    nbase = wid * NWN
    pltpu.sync_copy(typesP.at[pl.ds(nbase, NWN)], idxn_v)
    pltpu.async_copy(a_t.at[idxn_v], rows_a, sem).wait()
    pltpu.sync_copy(batchP.at[pl.ds(nbase, NWN)], idxn_v)
    pltpu.async_copy(bl_t.at[idxn_v], rows_b, sem).wait()

    def nbody(r, carry):
        for kk in range(D // 16):
            sl = pl.ds(kk * 16, 16)
            rows_a[r, sl] = rows_a[r, sl] + rows_b[r, sl]
        return carry

    lax.fori_loop(0, NWN, nbody, 0)
    pltpu.sync_copy(rows_a, hn0.at[pl.ds(nbase, NWN), :])

    pltpu.sync_copy(fc_flat, fc_v)
    z16 = jnp.zeros((16,), F32)

    def zout(kk, carry):
        out_v[pl.ds(kk * 16, 16)] = z16
        return carry

    lax.fori_loop(0, (CH * 16) // 16, zout, 0)
    iota = lax.iota(jnp.int32, 16)

    def ebody(i, carry):
        base = (wid + i * NW) * CH
        pltpu.sync_copy(src.at[pl.ds(base, CH)], idxs_v)
        pltpu.sync_copy(dst.at[pl.ds(base, CH)], idxd_v)
        pltpu.sync_copy(shifts.at[pl.ds(base * 3, CH * 3)], sh_v)

        def jbody(j, jcarry):
            sl = pl.ds(j * 16, 16)
            s16 = idxs_v[sl]
            d16 = idxd_v[sl]
            e16 = j * 16 + iota
            for comp in range(3):
                fs = plsc.load_gather(fc_v, [s16 * 3 + comp])
                fd = plsc.load_gather(fc_v, [d16 * 3 + comp])
                sh = plsc.load_gather(sh_v, [e16 * 3 + comp])
                dv = fd - fs + sh
                ti = dv.astype(jnp.int32).astype(F32)
                fl = jnp.where(dv < ti, ti - 1.0, ti)
                plsc.store_scatter(out_v, [e16 * 16 + comp], dv - fl)
            return jcarry

        lax.fori_loop(0, CH // 16, jbody, 0)
        pltpu.sync_copy(out_v, disp_flat.at[pl.ds(base * 16, CH * 16)])
        return carry

    lax.fori_loop(0, _nchunks(wid, NCHUNK), ebody, 0)


def _k1_call(typesP, batchP, a_t, bl_t, fcp, src, dst, shifts):
    return pl.kernel(
        _k1_body,
        out_type=(jax.ShapeDtypeStruct((NP, D), F32),
                  jax.ShapeDtypeStruct((E * 16,), F32)),
        mesh=_SC_MESH,
        compiler_params=pltpu.CompilerParams(needs_layout_passes=False),
        scratch_types=[
            pltpu.VMEM((NWN,), jnp.int32),
            pltpu.VMEM((NWN, D), F32),
            pltpu.VMEM((NWN, D), F32),
            pltpu.VMEM((N * 3,), F32),
            pltpu.VMEM((CH,), jnp.int32),
            pltpu.VMEM((CH,), jnp.int32),
            pltpu.VMEM((CH * 3,), F32),
            pltpu.VMEM((CH * 16,), F32),
            pltpu.SemaphoreType.DMA,
        ],
    )(typesP, batchP, a_t, bl_t, fcp, src, dst, shifts)


_SIN_C = (6.2831855, -41.3417, 81.60525, -76.70578, 42.057533, -15.085474,
          3.7785523, -0.6179781)
_COS_C = (1.0, -19.739208, 64.93939, -85.45682, 60.244595, -26.425692,
          7.8995357, -1.6978502, 0.24478738)


def _k2_body(disp_ref, e16_ref, fint_ref, w1_ref, w2_ref, eb_ref, he_ref):
    d16 = disp_ref[...]
    f0 = _dot(d16, e16_ref[...])
    x = f0 * fint_ref[...]
    t = (x + 0.5).astype(jnp.int32).astype(F32)
    u = x - t
    v = u * u
    sp = jnp.full_like(v, _SIN_C[-1])
    for c in _SIN_C[-2::-1]:
        sp = sp * v + c
    sp = sp * u
    cp = jnp.full_like(v, _COS_C[-1])
    for c in _COS_C[-2::-1]:
        cp = cp * v + c
    he_ref[...] = (_dot(sp, w1_ref[...]) + _dot(cp, w2_ref[...]) + eb_ref[...])


def _k2_call(dispP, e16, fint, w1, w2, eb):
    return pl.pallas_call(
        _k2_body,
        grid=(E // TE,),
        in_specs=[
            pl.BlockSpec((TE, 16), lambda i: (i, 0)),
            pl.BlockSpec((16, 3 * NF), lambda i: (0, 0)),
            pl.BlockSpec((1, 3 * NF), lambda i: (0, 0)),
            pl.BlockSpec((3 * NF, D), lambda i: (0, 0)),
            pl.BlockSpec((3 * NF, D), lambda i: (0, 0)),
            pl.BlockSpec((1, D), lambda i: (0, 0)),
        ],
        out_specs=pl.BlockSpec((TE, D), lambda i: (i, 0)),
        out_shape=jax.ShapeDtypeStruct((E, D), F32),
    )(dispP, e16, fint, w1, w2, eb)


def _k3_body(hn, src, dst, hs, hd, idx_v, rows_v, sem):
    wid = lax.axis_index("c") * 16 + lax.axis_index("s")

    def body(i, carry):
        base = (wid + i * NW) * CH
        pltpu.sync_copy(src.at[pl.ds(base, CH)], idx_v)
        pltpu.async_copy(hn.at[idx_v], rows_v, sem).wait()
        pltpu.sync_copy(rows_v, hs.at[pl.ds(base, CH), :])
        pltpu.sync_copy(dst.at[pl.ds(base, CH)], idx_v)
        pltpu.async_copy(hn.at[idx_v], rows_v, sem).wait()
        pltpu.sync_copy(rows_v, hd.at[pl.ds(base, CH), :])
        return carry

    lax.fori_loop(0, _nchunks(wid, NCHUNK), body, 0)


def _k3_call(hn, src, dst):
    return pl.kernel(
        _k3_body,
        out_type=(jax.ShapeDtypeStruct((E, D), F32),
                  jax.ShapeDtypeStruct((E, D), F32)),
        mesh=_SC_MESH,
        scratch_types=[
            pltpu.VMEM((CH,), jnp.int32),
            pltpu.VMEM((CH, D), F32),
            pltpu.SemaphoreType.DMA,
        ],
    )(hn, src, dst)


def _k4_body(hs_ref, hd_ref, he_ref, ssel_ref, wq1, wq2, bq, wk1, wk2, bk,
             wv1, wv2, bv, w_ref, ex_ref):
    hs = hs_ref[...]
    hd = hd_ref[...]
    he = he_ref[...]
    q = _dot(hs, wq1[...]) + _dot(he, wq2[...]) + bq[...]
    k = _dot(hd, wk1[...]) + _dot(he, wk2[...]) + bk[...]
    v = _dot(hd, wv1[...]) + _dot(he, wv2[...]) + bv[...]
    qk = q * k
    sb = _dot(qk, ssel_ref[...])
    exb = jnp.exp(jnp.minimum(sb, 60.0))
    w_ref[...] = exb * v
    ex_ref[...] = exb


def _k4_call(hs, hd, he, ssel, wq1, wq2, bq, wk1, wk2, bk, wv1, wv2, bv):
    wspec = pl.BlockSpec((D, D), lambda i: (0, 0))
    bspec = pl.BlockSpec((1, D), lambda i: (0, 0))
    espec = pl.BlockSpec((TE, D), lambda i: (i, 0))
    return pl.pallas_call(
        _k4_body,
        grid=(E // TE,),
        in_specs=[espec, espec, espec, wspec,
                  wspec, wspec, bspec, wspec, wspec, bspec,
                  wspec, wspec, bspec],
        out_specs=(pl.BlockSpec((TE, D), lambda i: (i, 0)),
                   pl.BlockSpec((TE, D), lambda i: (i, 0))),
        out_shape=(jax.ShapeDtypeStruct((E, D), F32),
                   jax.ShapeDtypeStruct((E, D), F32)),
    )(hs, hd, he, ssel, wq1, wq2, bq, wk1, wk2, bk, wv1, wv2, bv)


CH5 = 256
NCHUNK5 = E // CH5


def _k5_body(w, x2, src, u, dens, idx_v, idxr_v, w_v, acc, sem):
    cc = lax.axis_index("c")
    ss = lax.axis_index("s")
    wid = cc * 16 + ss
    z16 = jnp.zeros((16,), F32)
    iota = lax.iota(jnp.int32, 16)

    def _zero_wv():
        def zrow(r, carry):
            for kk in range(D // 16):
                w_v[r, pl.ds(kk * 16, 16)] = z16
            return carry

        lax.fori_loop(0, 128, zrow, 0)

    def _set_own_rows(kk):
        row0 = ss * (NP // 16) + kk * 128

        def seti(j, c2):
            idxr_v[pl.ds(j * 16, 16)] = row0 + j * 16 + iota
            return c2

        lax.fori_loop(0, 128 // 16, seti, 0)
        return row0

    def _zero_acc():
        def initkk(kk, carry):
            _set_own_rows(kk)
            pltpu.sync_copy(w_v.at[pl.ds(0, 128), :], acc.at[idxr_v])
            return carry

        lax.fori_loop(0, NP // 16 // 128, initkk, 0)

    def _accum(ref):
        def body(i, carry):
            base = (wid + i * NW) * CH5
            pltpu.sync_copy(src.at[pl.ds(base, CH5)], idx_v)
            pltpu.sync_copy(ref.at[pl.ds(base, CH5), :], w_v)
            pltpu.sync_copy(w_v, acc.at[idx_v], add=True)
            return carry

        lax.fori_loop(0, _nchunks(wid, NCHUNK5), body, 0)

    def _readback(out3):
        def outkk(kk, carry):
            row0 = _set_own_rows(kk)
            pltpu.async_copy(acc.at[idxr_v], w_v.at[pl.ds(0, 128), :],
                             sem).wait()
            pltpu.sync_copy(w_v.at[pl.ds(0, 128), :],
                            out3.at[cc, pl.ds(row0, 128), :])
            return carry

        lax.fori_loop(0, NP // 16 // 128, outkk, 0)

    _zero_wv()
    _zero_acc()
    plsc.subcore_barrier()
    _accum(w)
    plsc.subcore_barrier()
    _readback(u)
    _zero_wv()
    _zero_acc()
    plsc.subcore_barrier()
    _accum(x2)
    plsc.subcore_barrier()
    _readback(dens)


def _k5_call(w, x2, src):
    return pl.kernel(
        _k5_body,
        out_type=(jax.ShapeDtypeStruct((2, NP, D), F32),
                  jax.ShapeDtypeStruct((2, NP, D), F32)),
        mesh=_SC_MESH,
        compiler_params=pltpu.CompilerParams(needs_layout_passes=False),
        scratch_types=[
            pltpu.VMEM((CH5,), jnp.int32),
            pltpu.VMEM((128,), jnp.int32),
            pltpu.VMEM((CH5, D), F32),
            pltpu.VMEM_SHARED((NP, D), F32),
            pltpu.SemaphoreType.DMA,
        ],
    )(w, x2, src)


def _k6_body(hn_ref, u0_ref, u1_ref, d0_ref, d1_ref, ow, ob, n1g, n1b,
             f1, f1b, f2, f2b, n2g, n2b, out_ref):
    denb = d0_ref[...] + d1_ref[...]
    agg = (u0_ref[...] + u1_ref[...]) / (denb + 1e-16)
    o = _dotp(agg, ow[...]) + ob[...]
    x = hn_ref[...] + o
    h1 = _ln_tc(x, n1g[...], n1b[...])
    ffh = _dotp(h1, f1[...]) + f1b[...]
    gg = 0.5 * ffh * (1.0 + lax.erf(ffh * (1.0 / math.sqrt(2.0))))
    ff = _dotp(gg, f2[...]) + f2b[...]
    out_ref[...] = _ln_tc(h1 + ff, n2g[...], n2b[...])


def _k6_call(hn, u2, dens2, ow, ob, n1g, n1b, f1, f1b, f2, f2b, n2g, n2b):
    nspec = pl.BlockSpec((TN, D), lambda i: (i, 0))
    return pl.pallas_call(
        _k6_body,
        grid=(NP // TN,),
        in_specs=[
            nspec,
            pl.BlockSpec((TN, D), lambda i: (i, 0)),
            pl.BlockSpec((TN, D), lambda i: (i + NP // TN, 0)),
            pl.BlockSpec((TN, D), lambda i: (i, 0)),
            pl.BlockSpec((TN, D), lambda i: (i + NP // TN, 0)),
            pl.BlockSpec((D, D), lambda i: (0, 0)),
            pl.BlockSpec((1, D), lambda i: (0, 0)),
            pl.BlockSpec((1, D), lambda i: (0, 0)),
            pl.BlockSpec((1, D), lambda i: (0, 0)),
            pl.BlockSpec((D, 4 * D), lambda i: (0, 0)),
            pl.BlockSpec((1, 4 * D), lambda i: (0, 0)),
            pl.BlockSpec((4 * D, D), lambda i: (0, 0)),
            pl.BlockSpec((1, D), lambda i: (0, 0)),
            pl.BlockSpec((1, D), lambda i: (0, 0)),
            pl.BlockSpec((1, D), lambda i: (0, 0)),
        ],
        out_specs=nspec,
        out_shape=jax.ShapeDtypeStruct((NP, D), F32),
    )(hn, u2, u2, dens2, dens2, ow, ob, n1g, n1b, f1, f1b, f2, f2b, n2g, n2b)


def kernel(atom_types, lattices, batch_idx, frac_coords, edge_index,
           edge_unit_shifts, num_atoms, token_idx, atom_emb, node_w, node_b,
           edge_w, edge_b, wq_w, wq_b, wkv_w, wkv_b, out_w, out_b, n1_g, n1_b,
           f1_w, f1_b, f2_w, f2_b, n2_g, n2_b):
    src = edge_index[0].astype(jnp.int32)
    dst = edge_index[1].astype(jnp.int32)
    typesP = jnp.pad(atom_types.astype(jnp.int32), (0, NP - N))
    batchP = jnp.pad(batch_idx.astype(jnp.int32), (0, NP - N))
    fcp = frac_coords.astype(F32).reshape(-1)
    shifts = edge_unit_shifts.astype(F32).reshape(-1)
    embP = jnp.pad(atom_emb.astype(F32), ((0, D - MAXEL), (0, 0)))
    lat9 = lattices.astype(F32).reshape(B, 9)

    cols = jnp.arange(3 * NF, dtype=jnp.int32)
    e16 = (cols[None, :] // NF == jnp.arange(16, dtype=jnp.int32)[:, None]
           ).astype(F32)
    fint = (cols % NF).astype(F32)[None, :]
    dcol = jnp.arange(D, dtype=jnp.int32)
    ssel = jnp.where(dcol[:, None] // 32 == dcol[None, :] // 32,
                     INV_SQRT_HD, 0.0).astype(F32)

    a_t, bl_t = _prep_call(embP, node_w[:D], lat9, node_w[D:], node_b[None, :])
    hn0, disp_flat = _k1_call(typesP, batchP, a_t, bl_t, fcp, src, dst, shifts)
    h_e = _k2_call(disp_flat.reshape(E, 16), e16, fint, edge_w[:3 * NF],
                   edge_w[3 * NF:], edge_b[None, :])

    hn = hn0
    for li in range(2):
        hs_, hd_ = _k3_call(hn, src, dst)
        w_, exp_ = _k4_call(
            hs_, hd_, h_e, ssel,
            wq_w[li, :D], wq_w[li, D:], wq_b[li][None, :],
            wkv_w[li, :D, :D], wkv_w[li, D:, :D], wkv_b[li][None, :D],
            wkv_w[li, :D, D:], wkv_w[li, D:, D:], wkv_b[li][None, D:])
        u_, dens_ = _k5_call(w_, exp_, src)
        hn = _k6_call(
            hn, u_.reshape(2 * NP, D), dens_.reshape(2 * NP, D),
            out_w[li], out_b[li][None, :], n1_g[li][None, :], n1_b[li][None, :],
            f1_w[li], f1_b[li][None, :], f2_w[li], f2_b[li][None, :],
            n2_g[li][None, :], n2_b[li][None, :])
    return hn[:N]


# speedup vs baseline: 1.0277x; 1.0209x over previous
import functools
import math

import jax
import jax.numpy as jnp
from jax import lax
from jax.experimental import pallas as pl
from jax.experimental.pallas import tpu as pltpu
from jax.experimental.pallas import tpu_sc as plsc

N = 10000
E = 320000
B = 64
D = 128
NF = 64
H = 4
MAXEL = 100

NP = 10240
NW = 32
CH = 512
NCHUNK = E // CH
NWN = NP // NW
TE = 512
TN = 2048
INV_SQRT_HD = 1.0 / math.sqrt(D // H)
TWO_PI = 2.0 * math.pi
F32 = jnp.float32

_SC_MESH = plsc.VectorSubcoreMesh(core_axis_name="c", subcore_axis_name="s")


def _nchunks(wid, nchunk):
    rem = nchunk - NW * (nchunk // NW)
    return jnp.where(wid < rem, nchunk // NW + 1, nchunk // NW)


def _dotT(a, b):
    return lax.dot_general(a, b, (((0,), (0,)), ((), ())),
                           preferred_element_type=F32)


def _dot(a, b):
    return jnp.dot(a, b, preferred_element_type=F32)


def _ln_tc(x, g, b):
    m = jnp.mean(x, axis=-1, keepdims=True)
    v = jnp.mean((x - m) ** 2, axis=-1, keepdims=True)
    return (x - m) / jnp.sqrt(v + 1e-5) * g + b


def _prep_body(emb_ref, nw1_ref, lat_ref, nw2_ref, nb_ref, a_ref, bl_ref):
    a_ref[...] = _dot(emb_ref[...], nw1_ref[...])
    lat = lat_ref[...]
    cols = []
    for (i, k) in ((0, 0), (0, 1), (0, 2), (1, 1), (1, 2), (2, 2)):
        c = (lat[:, 0 + i:1 + i] * lat[:, 0 + k:1 + k]
             + lat[:, 3 + i:4 + i] * lat[:, 3 + k:4 + k]
             + lat[:, 6 + i:7 + i] * lat[:, 6 + k:7 + k])
        cols.append(c)
    lf = jnp.concatenate(cols, axis=1)
    bl_ref[...] = _dot(lf, nw2_ref[...]) + nb_ref[...]


def _prep_call(embP, nw1, lat9, nw2, nb):
    return pl.pallas_call(
        _prep_body,
        out_shape=(jax.ShapeDtypeStruct((D, D), F32),
                   jax.ShapeDtypeStruct((B, D), F32)),
    )(embP, nw1, lat9, nw2, nb)


def _k1_body(typesP, batchP, a_t, bl_t, fc_flat, src, dst, shifts, hn0, disp_flat,
             idxn_v, rows_a, rows_b, fc_v, idxs_v, idxd_v, sh_v, out_v, sem):
    cc = lax.axis_index("c")
    ss = lax.axis_index("s")
    wid = cc * 16 + ss

    nbase = wid * NWN
    pltpu.sync_copy(typesP.at[pl.ds(nbase, NWN)], idxn_v)
    pltpu.async_copy(a_t.at[idxn_v], rows_a, sem).wait()
    pltpu.sync_copy(batchP.at[pl.ds(nbase, NWN)], idxn_v)
    pltpu.async_copy(bl_t.at[idxn_v], rows_b, sem).wait()

    def nbody(r, carry):
        for kk in range(D // 16):
            sl = pl.ds(kk * 16, 16)
            rows_a[r, sl] = rows_a[r, sl] + rows_b[r, sl]
        return carry

    lax.fori_loop(0, NWN, nbody, 0)
    pltpu.sync_copy(rows_a, hn0.at[pl.ds(nbase, NWN), :])

    pltpu.sync_copy(fc_flat, fc_v)
    z16 = jnp.zeros((16,), F32)

    def zout(kk, carry):
        out_v[pl.ds(kk * 16, 16)] = z16
        return carry

    lax.fori_loop(0, (CH * 16) // 16, zout, 0)
    iota = lax.iota(jnp.int32, 16)

    def ebody(i, carry):
        base = (wid + i * NW) * CH
        pltpu.sync_copy(src.at[pl.ds(base, CH)], idxs_v)
        pltpu.sync_copy(dst.at[pl.ds(base, CH)], idxd_v)
        pltpu.sync_copy(shifts.at[pl.ds(base * 3, CH * 3)], sh_v)

        def jbody(j, jcarry):
            sl = pl.ds(j * 16, 16)
            s16 = idxs_v[sl]
            d16 = idxd_v[sl]
            e16 = j * 16 + iota
            for comp in range(3):
                fs = plsc.load_gather(fc_v, [s16 * 3 + comp])
                fd = plsc.load_gather(fc_v, [d16 * 3 + comp])
                sh = plsc.load_gather(sh_v, [e16 * 3 + comp])
                dv = fd - fs + sh
                ti = dv.astype(jnp.int32).astype(F32)
                fl = jnp.where(dv < ti, ti - 1.0, ti)
                plsc.store_scatter(out_v, [e16 * 16 + comp], dv - fl)
            return jcarry

        lax.fori_loop(0, CH // 16, jbody, 0)
        pltpu.sync_copy(out_v, disp_flat.at[pl.ds(base * 16, CH * 16)])
        return carry

    lax.fori_loop(0, _nchunks(wid, NCHUNK), ebody, 0)


def _k1_call(typesP, batchP, a_t, bl_t, fcp, src, dst, shifts):
    return pl.kernel(
        _k1_body,
        out_type=(jax.ShapeDtypeStruct((NP, D), F32),
                  jax.ShapeDtypeStruct((E * 16,), F32)),
        mesh=_SC_MESH,
        compiler_params=pltpu.CompilerParams(needs_layout_passes=False),
        scratch_types=[
            pltpu.VMEM((NWN,), jnp.int32),
            pltpu.VMEM((NWN, D), F32),
            pltpu.VMEM((NWN, D), F32),
            pltpu.VMEM((N * 3,), F32),
            pltpu.VMEM((CH,), jnp.int32),
            pltpu.VMEM((CH,), jnp.int32),
            pltpu.VMEM((CH * 3,), F32),
            pltpu.VMEM((CH * 16,), F32),
            pltpu.SemaphoreType.DMA,
        ],
    )(typesP, batchP, a_t, bl_t, fcp, src, dst, shifts)


_SIN_C = (6.2831855, -41.3417, 81.60525, -76.70578, 42.057533, -15.085474,
          3.7785523, -0.6179781)
_COS_C = (1.0, -19.739208, 64.93939, -85.45682, 60.244595, -26.425692,
          7.8995357, -1.6978502, 0.24478738)


def _k2_body(disp_ref, e16_ref, fint_ref, w1_ref, w2_ref, eb_ref, he_ref):
    d16 = disp_ref[...]
    f0 = _dot(d16, e16_ref[...])
    x = f0 * fint_ref[...]
    t = (x + 0.5).astype(jnp.int32).astype(F32)
    u = x - t
    v = u * u
    sp = jnp.full_like(v, _SIN_C[-1])
    for c in _SIN_C[-2::-1]:
        sp = sp * v + c
    sp = sp * u
    cp = jnp.full_like(v, _COS_C[-1])
    for c in _COS_C[-2::-1]:
        cp = cp * v + c
    he_ref[...] = (_dot(sp, w1_ref[...]) + _dot(cp, w2_ref[...]) + eb_ref[...])


def _k2_call(dispP, e16, fint, w1, w2, eb):
    return pl.pallas_call(
        _k2_body,
        grid=(E // TE,),
        in_specs=[
            pl.BlockSpec((TE, 16), lambda i: (i, 0)),
            pl.BlockSpec((16, 3 * NF), lambda i: (0, 0)),
            pl.BlockSpec((1, 3 * NF), lambda i: (0, 0)),
            pl.BlockSpec((3 * NF, D), lambda i: (0, 0)),
            pl.BlockSpec((3 * NF, D), lambda i: (0, 0)),
            pl.BlockSpec((1, D), lambda i: (0, 0)),
        ],
        out_specs=pl.BlockSpec((TE, D), lambda i: (i, 0)),
        out_shape=jax.ShapeDtypeStruct((E, D), F32),
    )(dispP, e16, fint, w1, w2, eb)


def _k3_body(hn, src, dst, hs, hd, idx_v, rows_v, sem):
    wid = lax.axis_index("c") * 16 + lax.axis_index("s")

    def body(i, carry):
        base = (wid + i * NW) * CH
        pltpu.sync_copy(src.at[pl.ds(base, CH)], idx_v)
        pltpu.async_copy(hn.at[idx_v], rows_v, sem).wait()
        pltpu.sync_copy(rows_v, hs.at[pl.ds(base, CH), :])
        pltpu.sync_copy(dst.at[pl.ds(base, CH)], idx_v)
        pltpu.async_copy(hn.at[idx_v], rows_v, sem).wait()
        pltpu.sync_copy(rows_v, hd.at[pl.ds(base, CH), :])
        return carry

    lax.fori_loop(0, _nchunks(wid, NCHUNK), body, 0)


def _k3_call(hn, src, dst):
    return pl.kernel(
        _k3_body,
        out_type=(jax.ShapeDtypeStruct((E, D), F32),
                  jax.ShapeDtypeStruct((E, D), F32)),
        mesh=_SC_MESH,
        scratch_types=[
            pltpu.VMEM((CH,), jnp.int32),
            pltpu.VMEM((CH, D), F32),
            pltpu.SemaphoreType.DMA,
        ],
    )(hn, src, dst)


def _k4_body(hs_ref, hd_ref, he_ref, ssel_ref, wq1, wq2, bq, wk1, wk2, bk,
             wv1, wv2, bv, w_ref, ex_ref):
    hs = hs_ref[...]
    hd = hd_ref[...]
    he = he_ref[...]
    q = _dot(hs, wq1[...]) + _dot(he, wq2[...]) + bq[...]
    k = _dot(hd, wk1[...]) + _dot(he, wk2[...]) + bk[...]
    v = _dot(hd, wv1[...]) + _dot(he, wv2[...]) + bv[...]
    qk = q * k
    sb = _dot(qk, ssel_ref[...])
    exb = jnp.exp(jnp.minimum(sb, 60.0))
    w_ref[...] = exb * v
    ex_ref[...] = exb


def _k4_call(hs, hd, he, ssel, wq1, wq2, bq, wk1, wk2, bk, wv1, wv2, bv):
    wspec = pl.BlockSpec((D, D), lambda i: (0, 0))
    bspec = pl.BlockSpec((1, D), lambda i: (0, 0))
    espec = pl.BlockSpec((TE, D), lambda i: (i, 0))
    return pl.pallas_call(
        _k4_body,
        grid=(E // TE,),
        in_specs=[espec, espec, espec, wspec,
                  wspec, wspec, bspec, wspec, wspec, bspec,
                  wspec, wspec, bspec],
        out_specs=(pl.BlockSpec((TE, D), lambda i: (i, 0)),
                   pl.BlockSpec((TE, D), lambda i: (i, 0))),
        out_shape=(jax.ShapeDtypeStruct((E, D), F32),
                   jax.ShapeDtypeStruct((E, D), F32)),
    )(hs, hd, he, ssel, wq1, wq2, bq, wk1, wk2, bk, wv1, wv2, bv)


CH5 = 256
NCHUNK5 = E // CH5


def _k5_body(w, x2, src, u, dens, idx_v, idxr_v, w_v, acc, sem):
    cc = lax.axis_index("c")
    ss = lax.axis_index("s")
    wid = cc * 16 + ss
    z16 = jnp.zeros((16,), F32)
    iota = lax.iota(jnp.int32, 16)

    def _zero_wv():
        def zrow(r, carry):
            for kk in range(D // 16):
                w_v[r, pl.ds(kk * 16, 16)] = z16
            return carry

        lax.fori_loop(0, 128, zrow, 0)

    def _set_own_rows(kk):
        row0 = ss * (NP // 16) + kk * 128

        def seti(j, c2):
            idxr_v[pl.ds(j * 16, 16)] = row0 + j * 16 + iota
            return c2

        lax.fori_loop(0, 128 // 16, seti, 0)
        return row0

    def _zero_acc():
        def initkk(kk, carry):
            _set_own_rows(kk)
            pltpu.sync_copy(w_v.at[pl.ds(0, 128), :], acc.at[idxr_v])
            return carry

        lax.fori_loop(0, NP // 16 // 128, initkk, 0)

    def _accum(ref):
        def body(i, carry):
            base = (wid + i * NW) * CH5
            pltpu.sync_copy(src.at[pl.ds(base, CH5)], idx_v)
            pltpu.sync_copy(ref.at[pl.ds(base, CH5), :], w_v)
            pltpu.sync_copy(w_v, acc.at[idx_v], add=True)
            return carry

        lax.fori_loop(0, _nchunks(wid, NCHUNK5), body, 0)

    def _readback(out3):
        def outkk(kk, carry):
            row0 = _set_own_rows(kk)
            pltpu.async_copy(acc.at[idxr_v], w_v.at[pl.ds(0, 128), :],
                             sem).wait()
            pltpu.sync_copy(w_v.at[pl.ds(0, 128), :],
                            out3.at[cc, pl.ds(row0, 128), :])
            return carry

        lax.fori_loop(0, NP // 16 // 128, outkk, 0)

    _zero_wv()
    _zero_acc()
    plsc.subcore_barrier()
    _accum(w)
    plsc.subcore_barrier()
    _readback(u)
    _zero_wv()
    _zero_acc()
    plsc.subcore_barrier()
    _accum(x2)
    plsc.subcore_barrier()
    _readback(dens)


def _k5_call(w, x2, src):
    return pl.kernel(
        _k5_body,
        out_type=(jax.ShapeDtypeStruct((2, NP, D), F32),
                  jax.ShapeDtypeStruct((2, NP, D), F32)),
        mesh=_SC_MESH,
        compiler_params=pltpu.CompilerParams(needs_layout_passes=False),
        scratch_types=[
            pltpu.VMEM((CH5,), jnp.int32),
            pltpu.VMEM((128,), jnp.int32),
            pltpu.VMEM((CH5, D), F32),
            pltpu.VMEM_SHARED((NP, D), F32),
            pltpu.SemaphoreType.DMA,
        ],
    )(w, x2, src)


def _k6_body(hn_ref, u0_ref, u1_ref, d0_ref, d1_ref, ow, ob, n1g, n1b,
             f1, f1b, f2, f2b, n2g, n2b, out_ref):
    denb = d0_ref[...] + d1_ref[...]
    agg = (u0_ref[...] + u1_ref[...]) / (denb + 1e-16)
    o = _dot(agg, ow[...]) + ob[...]
    x = hn_ref[...] + o
    h1 = _ln_tc(x, n1g[...], n1b[...])
    ffh = _dot(h1, f1[...]) + f1b[...]
    gg = 0.5 * ffh * (1.0 + lax.erf(ffh * (1.0 / math.sqrt(2.0))))
    ff = _dot(gg, f2[...]) + f2b[...]
    out_ref[...] = _ln_tc(h1 + ff, n2g[...], n2b[...])


def _k6_call(hn, u2, dens2, ow, ob, n1g, n1b, f1, f1b, f2, f2b, n2g, n2b):
    nspec = pl.BlockSpec((TN, D), lambda i: (i, 0))
    return pl.pallas_call(
        _k6_body,
        grid=(NP // TN,),
        in_specs=[
            nspec,
            pl.BlockSpec((TN, D), lambda i: (i, 0)),
            pl.BlockSpec((TN, D), lambda i: (i + NP // TN, 0)),
            pl.BlockSpec((TN, D), lambda i: (i, 0)),
            pl.BlockSpec((TN, D), lambda i: (i + NP // TN, 0)),
            pl.BlockSpec((D, D), lambda i: (0, 0)),
            pl.BlockSpec((1, D), lambda i: (0, 0)),
            pl.BlockSpec((1, D), lambda i: (0, 0)),
            pl.BlockSpec((1, D), lambda i: (0, 0)),
            pl.BlockSpec((D, 4 * D), lambda i: (0, 0)),
            pl.BlockSpec((1, 4 * D), lambda i: (0, 0)),
            pl.BlockSpec((4 * D, D), lambda i: (0, 0)),
            pl.BlockSpec((1, D), lambda i: (0, 0)),
            pl.BlockSpec((1, D), lambda i: (0, 0)),
            pl.BlockSpec((1, D), lambda i: (0, 0)),
        ],
        out_specs=nspec,
        out_shape=jax.ShapeDtypeStruct((NP, D), F32),
    )(hn, u2, u2, dens2, dens2, ow, ob, n1g, n1b, f1, f1b, f2, f2b, n2g, n2b)


def kernel(atom_types, lattices, batch_idx, frac_coords, edge_index,
           edge_unit_shifts, num_atoms, token_idx, atom_emb, node_w, node_b,
           edge_w, edge_b, wq_w, wq_b, wkv_w, wkv_b, out_w, out_b, n1_g, n1_b,
           f1_w, f1_b, f2_w, f2_b, n2_g, n2_b):
    src = edge_index[0].astype(jnp.int32)
    dst = edge_index[1].astype(jnp.int32)
    typesP = jnp.pad(atom_types.astype(jnp.int32), (0, NP - N))
    batchP = jnp.pad(batch_idx.astype(jnp.int32), (0, NP - N))
    fcp = frac_coords.astype(F32).reshape(-1)
    shifts = edge_unit_shifts.astype(F32).reshape(-1)
    embP = jnp.pad(atom_emb.astype(F32), ((0, D - MAXEL), (0, 0)))
    lat9 = lattices.astype(F32).reshape(B, 9)

    cols = jnp.arange(3 * NF, dtype=jnp.int32)
    e16 = (cols[None, :] // NF == jnp.arange(16, dtype=jnp.int32)[:, None]
           ).astype(F32)
    fint = (cols % NF).astype(F32)[None, :]
    dcol = jnp.arange(D, dtype=jnp.int32)
    ssel = jnp.where(dcol[:, None] // 32 == dcol[None, :] // 32,
                     INV_SQRT_HD, 0.0).astype(F32)

    a_t, bl_t = _prep_call(embP, node_w[:D], lat9, node_w[D:], node_b[None, :])
    hn0, disp_flat = _k1_call(typesP, batchP, a_t, bl_t, fcp, src, dst, shifts)
    h_e = _k2_call(disp_flat.reshape(E, 16), e16, fint, edge_w[:3 * NF],
                   edge_w[3 * NF:], edge_b[None, :])

    hn = hn0
    for li in range(2):
        hs_, hd_ = _k3_call(hn, src, dst)
        w_, exp_ = _k4_call(
            hs_, hd_, h_e, ssel,
            wq_w[li, :D], wq_w[li, D:], wq_b[li][None, :],
            wkv_w[li, :D, :D], wkv_w[li, D:, :D], wkv_b[li][None, :D],
            wkv_w[li, :D, D:], wkv_w[li, D:, D:], wkv_b[li][None, D:])
        u_, dens_ = _k5_call(w_, exp_, src)
        hn = _k6_call(
            hn, u_.reshape(2 * NP, D), dens_.reshape(2 * NP, D),
            out_w[li], out_b[li][None, :], n1_g[li][None, :], n1_b[li][None, :],
            f1_w[li], f1_b[li][None, :], f2_w[li], f2_b[li][None, :],
            n2_g[li][None, :], n2_b[li][None, :])
    return hn[:N]
